# initial kernel scaffold (unmeasured)
import jax
import jax.numpy as jnp
from jax import lax
from jax.experimental import pallas as pl
from jax.experimental.pallas import tpu as pltpu

N_DEV = 8
N_EXP = 32
E_LOC = 4
CAP = 409
T = 2048
D = 512
H = 1024


def _cumsum0(a, n_rows):
    sh = 1
    while sh < n_rows:
        z = jnp.zeros((sh, a.shape[1]), a.dtype)
        a = a + jnp.concatenate([z, a[:-sh, :]], axis=0)
        sh *= 2
    return a


def kernel(x, router_W, route_idx, expert_W):
    del router_W
    x_bf = x.astype(jnp.bfloat16)
    w_bf = expert_W.astype(jnp.bfloat16)

    def body(x_ref, ridx_ref, w_ref, out_ref, wbuf, cnts, wssem, wrsem, cssem, crsem):
        me = lax.axis_index("i")
        left = lax.rem(me - 1 + N_DEV, N_DEV)
        right = lax.rem(me + 1, N_DEV)

        ridx = ridx_ref[:, :]
        eids = lax.broadcasted_iota(jnp.int32, (T, N_EXP), 1)
        oh = (ridx == eids).astype(jnp.int32)
        cnts[pl.ds(me, 1), :] = jnp.sum(oh, axis=0, keepdims=True)

        bar = pltpu.get_barrier_semaphore()
        for nbr in (left, right):
            pl.semaphore_signal(bar, inc=1, device_id=(nbr,),
                                device_id_type=pl.DeviceIdType.MESH)
        pl.semaphore_wait(bar, 2)

        for h in range(1, N_DEV):
            row = lax.rem(me - h + 1 + N_DEV, N_DEV)
            rdma = pltpu.make_async_remote_copy(
                src_ref=cnts.at[pl.ds(row, 1)],
                dst_ref=cnts.at[pl.ds(row, 1)],
                send_sem=cssem.at[h - 1],
                recv_sem=crsem.at[h - 1],
                device_id=(right,),
                device_id_type=pl.DeviceIdType.MESH,
            )
            rdma.start()
            rdma.wait()

        srows = lax.broadcasted_iota(jnp.int32, (N_DEV, N_EXP), 0)
        prefix = jnp.sum(jnp.where(srows < me, cnts[:, :], 0),
                         axis=0, keepdims=True)
        excl = _cumsum0(oh, T) - oh
        local_rank = jnp.sum(oh * excl, axis=1, keepdims=True)
        tok_prefix = jnp.sum(oh * prefix, axis=1, keepdims=True)
        keep = (local_rank + tok_prefix) < CAP

        x_loc = x_ref[:, :]

        def accum(is_first, e, w):
            m = jnp.logical_and(ridx == e, keep)
            xm = x_loc * m.astype(jnp.bfloat16)
            contrib = jnp.dot(xm, w, preferred_element_type=jnp.float32)
            if is_first:
                out_ref[:, :] = contrib
            else:
                out_ref[:, :] = out_ref[:, :] + contrib

        for h in range(1, N_DEV):
            origin = lax.rem(me - h + 1 + N_DEV, N_DEV)
            src = w_ref if h == 1 else wbuf.at[h - 2]
            rdma = pltpu.make_async_remote_copy(
                src_ref=src,
                dst_ref=wbuf.at[h - 1],
                send_sem=wssem.at[h - 1],
                recv_sem=wrsem.at[h - 1],
                device_id=(right,),
                device_id_type=pl.DeviceIdType.MESH,
            )
            rdma.start()
            for k in range(E_LOC):
                w = w_ref[k] if h == 1 else wbuf[h - 2, k]
                accum(h == 1 and k == 0, origin * E_LOC + k, w)
            rdma.wait()
        for k in range(E_LOC):
            accum(False, right * E_LOC + k, wbuf[N_DEV - 2, k])

    return pl.pallas_call(
        body,
        out_shape=jax.ShapeDtypeStruct((T, H), jnp.float32),
        in_specs=[
            pl.BlockSpec(memory_space=pltpu.VMEM),
            pl.BlockSpec(memory_space=pltpu.VMEM),
            pl.BlockSpec(memory_space=pltpu.VMEM),
        ],
        out_specs=pl.BlockSpec(memory_space=pltpu.VMEM),
        scratch_shapes=[
            pltpu.VMEM((N_DEV - 1, E_LOC, D, H), jnp.bfloat16),
            pltpu.VMEM((N_DEV, N_EXP), jnp.int32),
            pltpu.SemaphoreType.DMA((N_DEV - 1,)),
            pltpu.SemaphoreType.DMA((N_DEV - 1,)),
            pltpu.SemaphoreType.DMA((N_DEV - 1,)),
            pltpu.SemaphoreType.DMA((N_DEV - 1,)),
        ],
        compiler_params=pltpu.CompilerParams(collective_id=0),
    )(x_bf, route_idx, w_bf)


# baseline (device time: 378138 ns/iter reference)
import jax
import jax.numpy as jnp
from jax import lax
from jax.experimental import pallas as pl
from jax.experimental.pallas import tpu as pltpu

N_DEV = 8
N_EXP = 32
E_LOC = 4
CAP = 409
T = 2048
D = 512
H = 1024


def _cumsum0(a, n_rows):
    sh = 1
    while sh < n_rows:
        z = jnp.zeros((sh, a.shape[1]), a.dtype)
        a = a + jnp.concatenate([z, a[:-sh, :]], axis=0)
        sh *= 2
    return a


def kernel(x, router_W, route_idx, expert_W):
    del router_W
    x_bf = x.astype(jnp.bfloat16)
    w_bf = expert_W.astype(jnp.bfloat16)

    def body(x_ref, ridx_ref, w_ref, out_ref, wbuf, cnts, wssem, wrsem, cssem, crsem):
        me = lax.axis_index("i")
        left = lax.rem(me - 1 + N_DEV, N_DEV)
        right = lax.rem(me + 1, N_DEV)

        ridx = ridx_ref[:, :]
        eids = lax.broadcasted_iota(jnp.int32, (T, N_EXP), 1)
        oh = (ridx == eids).astype(jnp.int32)
        cnts[pl.ds(me, 1), :] = jnp.sum(oh, axis=0, keepdims=True)

        bar = pltpu.get_barrier_semaphore()
        for nbr in (left, right):
            pl.semaphore_signal(bar, inc=1, device_id=(nbr,),
                                device_id_type=pl.DeviceIdType.MESH)
        pl.semaphore_wait(bar, 2)

        for h in range(1, N_DEV):
            row = lax.rem(me - h + 1 + N_DEV, N_DEV)
            rdma = pltpu.make_async_remote_copy(
                src_ref=cnts.at[pl.ds(row, 1)],
                dst_ref=cnts.at[pl.ds(row, 1)],
                send_sem=cssem.at[h - 1],
                recv_sem=crsem.at[h - 1],
                device_id=(right,),
                device_id_type=pl.DeviceIdType.MESH,
            )
            rdma.start()
            rdma.wait()

        srows = lax.broadcasted_iota(jnp.int32, (N_DEV, N_EXP), 0)
        prefix = jnp.sum(jnp.where(srows < me, cnts[:, :], 0),
                         axis=0, keepdims=True)
        excl = _cumsum0(oh, T) - oh
        local_rank = jnp.sum(oh * excl, axis=1, keepdims=True)
        tok_prefix = jnp.sum(oh * prefix, axis=1, keepdims=True)
        keep = (local_rank + tok_prefix) < CAP

        x_loc = x_ref[:, :]

        def accum(is_first, e, w):
            m = jnp.logical_and(ridx == e, keep)
            xm = x_loc * m.astype(jnp.bfloat16)
            contrib = jnp.dot(xm, w, preferred_element_type=jnp.float32)
            if is_first:
                out_ref[:, :] = contrib
            else:
                out_ref[:, :] = out_ref[:, :] + contrib

        for h in range(1, N_DEV):
            origin = lax.rem(me - h + 1 + N_DEV, N_DEV)
            src = w_ref if h == 1 else wbuf.at[h - 2]
            rdma = pltpu.make_async_remote_copy(
                src_ref=src,
                dst_ref=wbuf.at[h - 1],
                send_sem=wssem.at[h - 1],
                recv_sem=wrsem.at[h - 1],
                device_id=(right,),
                device_id_type=pl.DeviceIdType.MESH,
            )
            rdma.start()
            for k in range(E_LOC):
                w = w_ref[k] if h == 1 else wbuf[h - 2, k]
                accum(h == 1 and k == 0, origin * E_LOC + k, w)
            rdma.wait()
        for k in range(E_LOC):
            accum(False, right * E_LOC + k, wbuf[N_DEV - 2, k])

    return pl.pallas_call(
        body,
        out_shape=jax.ShapeDtypeStruct((T, H), jnp.float32),
        in_specs=[
            pl.BlockSpec(memory_space=pltpu.VMEM),
            pl.BlockSpec(memory_space=pltpu.VMEM),
            pl.BlockSpec(memory_space=pltpu.VMEM),
        ],
        out_specs=pl.BlockSpec(memory_space=pltpu.VMEM),
        scratch_shapes=[
            pltpu.VMEM((N_DEV - 1, E_LOC, D, H), jnp.bfloat16),
            pltpu.VMEM((N_DEV, N_EXP), jnp.int32),
            pltpu.SemaphoreType.DMA((N_DEV - 1,)),
            pltpu.SemaphoreType.DMA((N_DEV - 1,)),
            pltpu.SemaphoreType.DMA((N_DEV - 1,)),
            pltpu.SemaphoreType.DMA((N_DEV - 1,)),
        ],
        compiler_params=pltpu.CompilerParams(
            collective_id=0,
            vmem_limit_bytes=60 * 1024 * 1024,
        ),
    )(x_bf, route_idx, w_bf)


# device time: 235835 ns/iter; 1.6034x vs baseline; 1.6034x over previous
import jax
import jax.numpy as jnp
from jax import lax
from jax.experimental import pallas as pl
from jax.experimental.pallas import tpu as pltpu

N_DEV = 8
N_EXP = 32
E_LOC = 4
CAP = 409
T = 2048
D = 512
H = 1024

R_HOPS = 4
L_HOPS = 3


def _cumsum0(a, n_rows):
    sh = 1
    while sh < n_rows:
        z = jnp.zeros((sh, a.shape[1]), a.dtype)
        a = a + jnp.concatenate([z, a[:-sh, :]], axis=0)
        sh *= 2
    return a


def kernel(x, router_W, route_idx, expert_W):
    del router_W
    x_bf = x.astype(jnp.bfloat16)
    w_bf = expert_W.astype(jnp.bfloat16)

    def body(x_ref, ridx_ref, w_ref, out_ref,
             rbuf, lbuf, cnts,
             rssem, rrsem, lssem, lrsem,
             crssem, crrsem, clssem, clrsem):
        me = lax.axis_index("i")
        left = lax.rem(me - 1 + N_DEV, N_DEV)
        right = lax.rem(me + 1, N_DEV)

        def copy(src, dst, ssem, rsem, dev):
            return pltpu.make_async_remote_copy(
                src_ref=src, dst_ref=dst, send_sem=ssem, recv_sem=rsem,
                device_id=(dev,), device_id_type=pl.DeviceIdType.MESH,
            )

        ridx = ridx_ref[:, :]
        eids = lax.broadcasted_iota(jnp.int32, (T, N_EXP), 1)
        oh = (ridx == eids).astype(jnp.int32)
        cnts[pl.ds(me, 1), :] = jnp.sum(oh, axis=0, keepdims=True)

        bar = pltpu.get_barrier_semaphore()
        for nbr in (left, right):
            pl.semaphore_signal(bar, inc=1, device_id=(nbr,),
                                device_id_type=pl.DeviceIdType.MESH)
        pl.semaphore_wait(bar, 2)

        def crow(o):
            return cnts.at[pl.ds(lax.rem(o + 2 * N_DEV, N_DEV), 1)]

        for r in range(1, R_HOPS + 1):
            rdmas = [copy(crow(me - r + 1), crow(me - r + 1),
                          crssem.at[r - 1], crrsem.at[r - 1], right)]
            if r <= L_HOPS:
                rdmas.append(copy(crow(me + r - 1), crow(me + r - 1),
                                  clssem.at[r - 1], clrsem.at[r - 1], left))
            for rd in rdmas:
                rd.start()
            for rd in rdmas:
                rd.wait()

        srows = lax.broadcasted_iota(jnp.int32, (N_DEV, N_EXP), 0)
        prefix = jnp.sum(jnp.where(srows < me, cnts[:, :], 0),
                         axis=0, keepdims=True)
        excl = _cumsum0(oh, T) - oh
        local_rank = jnp.sum(oh * excl, axis=1, keepdims=True)
        tok_prefix = jnp.sum(oh * prefix, axis=1, keepdims=True)
        keep = (local_rank + tok_prefix) < CAP

        def accum(is_first, e, w):
            m = jnp.logical_and(ridx == e, keep).astype(jnp.bfloat16)
            half = T // 2
            for t0 in (0, half):
                sl = pl.ds(t0, half)
                xm = x_ref[sl, :] * m[t0:t0 + half]
                contrib = jnp.dot(xm, w, preferred_element_type=jnp.float32)
                if is_first:
                    out_ref[sl, :] = contrib
                else:
                    out_ref[sl, :] = out_ref[sl, :] + contrib

        def accum_block(is_first, origin, wblock):
            o = lax.rem(origin + 2 * N_DEV, N_DEV)
            for k in range(E_LOC):
                accum(is_first and k == 0, o * E_LOC + k, wblock[k])

        for r in range(1, R_HOPS + 1):
            rsrc = w_ref if r == 1 else rbuf.at[r - 2]
            rdst = lbuf.at[0] if r == R_HOPS else rbuf.at[r - 1]
            rdmas = [copy(rsrc, rdst,
                          rssem.at[r - 1], rrsem.at[r - 1], right)]
            if r <= L_HOPS:
                lsrc = w_ref if r == 1 else lbuf.at[r - 2]
                rdmas.append(copy(lsrc, lbuf.at[r - 1],
                                  lssem.at[r - 1], lrsem.at[r - 1], left))
            for rd in rdmas:
                rd.start()
            if r == 1:
                accum_block(True, me, w_ref)
            else:
                accum_block(False, me - (r - 1), rbuf[r - 2])
                accum_block(False, me + (r - 1), lbuf[r - 2])
            for rd in rdmas:
                rd.wait()
        accum_block(False, me - R_HOPS, lbuf[0])

    return pl.pallas_call(
        body,
        out_shape=jax.ShapeDtypeStruct((T, H), jnp.float32),
        in_specs=[
            pl.BlockSpec(memory_space=pltpu.VMEM),
            pl.BlockSpec(memory_space=pltpu.VMEM),
            pl.BlockSpec(memory_space=pltpu.VMEM),
        ],
        out_specs=pl.BlockSpec(memory_space=pltpu.VMEM),
        scratch_shapes=[
            pltpu.VMEM((R_HOPS - 1, E_LOC, D, H), jnp.bfloat16),
            pltpu.VMEM((L_HOPS, E_LOC, D, H), jnp.bfloat16),
            pltpu.VMEM((N_DEV, N_EXP), jnp.int32),
            pltpu.SemaphoreType.DMA((R_HOPS,)),
            pltpu.SemaphoreType.DMA((R_HOPS,)),
            pltpu.SemaphoreType.DMA((L_HOPS,)),
            pltpu.SemaphoreType.DMA((L_HOPS,)),
            pltpu.SemaphoreType.DMA((R_HOPS,)),
            pltpu.SemaphoreType.DMA((R_HOPS,)),
            pltpu.SemaphoreType.DMA((L_HOPS,)),
            pltpu.SemaphoreType.DMA((L_HOPS,)),
        ],
        compiler_params=pltpu.CompilerParams(
            collective_id=0,
            vmem_limit_bytes=63 * 1024 * 1024,
        ),
    )(x_bf, route_idx, w_bf)


# device time: 207732 ns/iter; 1.8203x vs baseline; 1.1353x over previous
import jax
import jax.numpy as jnp
from jax import lax
from jax.experimental import pallas as pl
from jax.experimental.pallas import tpu as pltpu

N_DEV = 8
N_EXP = 32
E_LOC = 4
CAP = 409
T = 2048
D = 512
H = 1024

R_HOPS = 4
L_HOPS = 3


def _cumsum0(a, n_rows):
    sh = 1
    while sh < n_rows:
        z = jnp.zeros((sh, a.shape[1]), a.dtype)
        a = a + jnp.concatenate([z, a[:-sh, :]], axis=0)
        sh *= 2
    return a


def kernel(x, router_W, route_idx, expert_W):
    del router_W
    x_bf = x.astype(jnp.bfloat16)
    w_bf = expert_W.astype(jnp.bfloat16)

    def body(x_ref, ridx_ref, w_ref, out_ref,
             rbuf, lbuf, cnts,
             rssem, rrsem, lssem, lrsem,
             cssem, crsem):
        me = lax.axis_index("i")
        left = lax.rem(me - 1 + N_DEV, N_DEV)
        right = lax.rem(me + 1, N_DEV)

        def copy(src, dst, ssem, rsem, dev):
            return pltpu.make_async_remote_copy(
                src_ref=src, dst_ref=dst, send_sem=ssem, recv_sem=rsem,
                device_id=(dev,), device_id_type=pl.DeviceIdType.MESH,
            )

        ridx = ridx_ref[:, :]
        eids = lax.broadcasted_iota(jnp.int32, (T, N_EXP), 1)
        oh = (ridx == eids).astype(jnp.int32)
        cnts[pl.ds(me, 1), :] = jnp.sum(oh, axis=0, keepdims=True)

        bar = pltpu.get_barrier_semaphore()
        for j in range(1, N_DEV):
            pl.semaphore_signal(bar, inc=1,
                                device_id=(lax.rem(me + j, N_DEV),),
                                device_id_type=pl.DeviceIdType.MESH)
        pl.semaphore_wait(bar, N_DEV - 1)

        myrow = cnts.at[pl.ds(me, 1)]
        crdmas = [copy(myrow, myrow, cssem.at[j - 1], crsem.at[j - 1],
                       lax.rem(me + j, N_DEV))
                  for j in range(1, N_DEV)]
        for rd in crdmas:
            rd.start()
        for rd in crdmas:
            rd.wait()

        r1 = [copy(w_ref, rbuf.at[0], rssem.at[0], rrsem.at[0], right),
              copy(w_ref, lbuf.at[0], lssem.at[0], lrsem.at[0], left)]
        for rd in r1:
            rd.start()

        srows = lax.broadcasted_iota(jnp.int32, (N_DEV, N_EXP), 0)
        prefix = jnp.sum(jnp.where(srows < me, cnts[:, :], 0),
                         axis=0, keepdims=True)
        excl = _cumsum0(oh, T) - oh
        local_rank = jnp.sum(oh * excl, axis=1, keepdims=True)
        tok_prefix = jnp.sum(oh * prefix, axis=1, keepdims=True)
        keep = (local_rank + tok_prefix) < CAP

        def accum(is_first, e, w):
            m = jnp.logical_and(ridx == e, keep).astype(jnp.bfloat16)
            half = T // 2
            for t0 in (0, half):
                sl = pl.ds(t0, half)
                xm = x_ref[sl, :] * m[t0:t0 + half]
                contrib = jnp.dot(xm, w, preferred_element_type=jnp.float32)
                if is_first:
                    out_ref[sl, :] = contrib
                else:
                    out_ref[sl, :] = out_ref[sl, :] + contrib

        def accum_block(is_first, origin, wblock):
            o = lax.rem(origin + 2 * N_DEV, N_DEV)
            for k in range(E_LOC):
                accum(is_first and k == 0, o * E_LOC + k, wblock[k])

        accum_block(True, me, w_ref)
        for rd in r1:
            rd.wait()
        for r in (2, 3):
            rdmas = [copy(rbuf.at[r - 2], rbuf.at[r - 1],
                          rssem.at[r - 1], rrsem.at[r - 1], right),
                     copy(lbuf.at[r - 2], lbuf.at[r - 1],
                          lssem.at[r - 1], lrsem.at[r - 1], left)]
            for rd in rdmas:
                rd.start()
            accum_block(False, me - (r - 1), rbuf[r - 2])
            accum_block(False, me + (r - 1), lbuf[r - 2])
            for rd in rdmas:
                rd.wait()
        rdmas = [copy(rbuf.at[2, pl.ds(0, 2)], lbuf.at[0, pl.ds(0, 2)],
                      rssem.at[3], rrsem.at[3], right),
                 copy(lbuf.at[2, pl.ds(2, 2)], lbuf.at[0, pl.ds(2, 2)],
                      lssem.at[3], lrsem.at[3], left)]
        for rd in rdmas:
            rd.start()
        accum_block(False, me - 3, rbuf[2])
        accum_block(False, me + 3, lbuf[2])
        for rd in rdmas:
            rd.wait()
        accum_block(False, me - 4, lbuf[0])

    return pl.pallas_call(
        body,
        out_shape=jax.ShapeDtypeStruct((T, H), jnp.float32),
        in_specs=[
            pl.BlockSpec(memory_space=pltpu.VMEM),
            pl.BlockSpec(memory_space=pltpu.VMEM),
            pl.BlockSpec(memory_space=pltpu.VMEM),
        ],
        out_specs=pl.BlockSpec(memory_space=pltpu.VMEM),
        scratch_shapes=[
            pltpu.VMEM((R_HOPS - 1, E_LOC, D, H), jnp.bfloat16),
            pltpu.VMEM((L_HOPS, E_LOC, D, H), jnp.bfloat16),
            pltpu.VMEM((N_DEV, N_EXP), jnp.int32),
            pltpu.SemaphoreType.DMA((R_HOPS,)),
            pltpu.SemaphoreType.DMA((R_HOPS,)),
            pltpu.SemaphoreType.DMA((R_HOPS,)),
            pltpu.SemaphoreType.DMA((R_HOPS,)),
            pltpu.SemaphoreType.DMA((N_DEV - 1,)),
            pltpu.SemaphoreType.DMA((N_DEV - 1,)),
        ],
        compiler_params=pltpu.CompilerParams(
            collective_id=0,
            vmem_limit_bytes=63 * 1024 * 1024,
        ),
    )(x_bf, route_idx, w_bf)


# device time: 207682 ns/iter; 1.8208x vs baseline; 1.0002x over previous
import jax
import jax.numpy as jnp
from jax import lax
from jax.experimental import pallas as pl
from jax.experimental.pallas import tpu as pltpu

N_DEV = 8
N_EXP = 32
E_LOC = 4
CAP = 409
T = 2048
D = 512
H = 1024

R_HOPS = 4
L_HOPS = 3


def _cumsum0(a, n_rows):
    sh = 1
    while sh < n_rows:
        z = jnp.zeros((sh, a.shape[1]), a.dtype)
        a = a + jnp.concatenate([z, a[:-sh, :]], axis=0)
        sh *= 2
    return a


def kernel(x, router_W, route_idx, expert_W):
    del router_W
    x_bf = x.astype(jnp.bfloat16)
    w_bf = expert_W.astype(jnp.bfloat16)

    def body(x_ref, ridx_ref, w_ref, out_ref,
             rbuf, lbuf, cnts,
             rssem, rrsem, lssem, lrsem,
             cssem, crsem):
        me = lax.axis_index("i")
        left = lax.rem(me - 1 + N_DEV, N_DEV)
        right = lax.rem(me + 1, N_DEV)

        def copy(src, dst, ssem, rsem, dev):
            return pltpu.make_async_remote_copy(
                src_ref=src, dst_ref=dst, send_sem=ssem, recv_sem=rsem,
                device_id=(dev,), device_id_type=pl.DeviceIdType.MESH,
            )

        ridx = ridx_ref[:, :]
        eids = lax.broadcasted_iota(jnp.int32, (T, N_EXP), 1)
        oh = (ridx == eids).astype(jnp.int32)
        cnts[pl.ds(me, 1), :] = jnp.sum(oh, axis=0, keepdims=True)

        bar = pltpu.get_barrier_semaphore()
        for j in range(1, N_DEV):
            pl.semaphore_signal(bar, inc=1,
                                device_id=(lax.rem(me + j, N_DEV),),
                                device_id_type=pl.DeviceIdType.MESH)
        pl.semaphore_wait(bar, N_DEV - 1)

        myrow = cnts.at[pl.ds(me, 1)]
        crdmas = [copy(myrow, myrow, cssem.at[j - 1], crsem.at[j - 1],
                       lax.rem(me + j, N_DEV))
                  for j in range(1, N_DEV)]
        for rd in crdmas:
            rd.start()
        for rd in crdmas:
            rd.wait()

        r1 = [copy(w_ref, rbuf.at[0], rssem.at[0], rrsem.at[0], right),
              copy(w_ref, lbuf.at[0], lssem.at[0], lrsem.at[0], left)]
        for rd in r1:
            rd.start()

        srows = lax.broadcasted_iota(jnp.int32, (N_DEV, N_EXP), 0)
        prefix = jnp.sum(jnp.where(srows < me, cnts[:, :], 0),
                         axis=0, keepdims=True)
        excl = _cumsum0(oh, T) - oh
        local_rank = jnp.sum(oh * excl, axis=1, keepdims=True)
        tok_prefix = jnp.sum(oh * prefix, axis=1, keepdims=True)
        keep = (local_rank + tok_prefix) < CAP

        def accum_block(is_first, origin, wblock):
            o = lax.rem(origin + 2 * N_DEV, N_DEV)
            masks = [jnp.logical_and(ridx == o * E_LOC + k, keep)
                     .astype(jnp.bfloat16) for k in range(E_LOC)]
            w_cat = wblock.reshape(E_LOC * D, H)
            q = T // 4
            for t0 in range(0, T, q):
                sl = pl.ds(t0, q)
                xm_cat = jnp.concatenate(
                    [x_ref[sl, :] * m[t0:t0 + q] for m in masks], axis=1)
                contrib = jnp.dot(xm_cat, w_cat,
                                  preferred_element_type=jnp.float32)
                if is_first:
                    out_ref[sl, :] = contrib
                else:
                    out_ref[sl, :] = out_ref[sl, :] + contrib

        accum_block(True, me, w_ref[:, :, :])
        for rd in r1:
            rd.wait()
        for r in (2, 3):
            rdmas = [copy(rbuf.at[r - 2], rbuf.at[r - 1],
                          rssem.at[r - 1], rrsem.at[r - 1], right),
                     copy(lbuf.at[r - 2], lbuf.at[r - 1],
                          lssem.at[r - 1], lrsem.at[r - 1], left)]
            for rd in rdmas:
                rd.start()
            accum_block(False, me - (r - 1), rbuf[r - 2])
            accum_block(False, me + (r - 1), lbuf[r - 2])
            for rd in rdmas:
                rd.wait()
        rdmas = [copy(rbuf.at[2, pl.ds(0, 2)], lbuf.at[0, pl.ds(0, 2)],
                      rssem.at[3], rrsem.at[3], right),
                 copy(lbuf.at[2, pl.ds(2, 2)], lbuf.at[0, pl.ds(2, 2)],
                      lssem.at[3], lrsem.at[3], left)]
        for rd in rdmas:
            rd.start()
        accum_block(False, me - 3, rbuf[2])
        accum_block(False, me + 3, lbuf[2])
        for rd in rdmas:
            rd.wait()
        accum_block(False, me - 4, lbuf[0])

    return pl.pallas_call(
        body,
        out_shape=jax.ShapeDtypeStruct((T, H), jnp.float32),
        in_specs=[
            pl.BlockSpec(memory_space=pltpu.VMEM),
            pl.BlockSpec(memory_space=pltpu.VMEM),
            pl.BlockSpec(memory_space=pltpu.VMEM),
        ],
        out_specs=pl.BlockSpec(memory_space=pltpu.VMEM),
        scratch_shapes=[
            pltpu.VMEM((R_HOPS - 1, E_LOC, D, H), jnp.bfloat16),
            pltpu.VMEM((L_HOPS, E_LOC, D, H), jnp.bfloat16),
            pltpu.VMEM((N_DEV, N_EXP), jnp.int32),
            pltpu.SemaphoreType.DMA((R_HOPS,)),
            pltpu.SemaphoreType.DMA((R_HOPS,)),
            pltpu.SemaphoreType.DMA((R_HOPS,)),
            pltpu.SemaphoreType.DMA((R_HOPS,)),
            pltpu.SemaphoreType.DMA((N_DEV - 1,)),
            pltpu.SemaphoreType.DMA((N_DEV - 1,)),
        ],
        compiler_params=pltpu.CompilerParams(
            collective_id=0,
            vmem_limit_bytes=63 * 1024 * 1024,
        ),
    )(x_bf, route_idx, w_bf)
